# Initial kernel scaffold; baseline (speedup 1.0000x reference)
import jax
import jax.numpy as jnp
from jax.experimental import pallas as pl


def kernel(player, embed_table, W_cont, b_cont):
    raise NotImplementedError("write your pallas kernel here")


# speedup vs baseline: 23.6919x; 23.6919x over previous
import jax
import jax.numpy as jnp
from jax.experimental import pallas as pl

HIDDEN = 512
B = 16384
NFEAT = 47
VOCAB = 128
EMB = HIDDEN // 4
BLK = 256


def _body(p_ref, tab_ref, w_ref, b_ref, o_ref):
    i = pl.program_id(0)
    p = p_ref[...]
    rows = jax.lax.broadcasted_iota(jnp.int32, (BLK, NFEAT), 0) + i * BLK
    p = jnp.where(rows >= B - 2, p // 10, p)
    tab = tab_ref[...]
    acc = jnp.full((BLK, EMB), -jnp.inf, jnp.float32)
    for f in range(NFEAT):
        col = jax.lax.slice(p, (0, f), (BLK, f + 1))
        oh = (col == jax.lax.broadcasted_iota(jnp.int32, (BLK, VOCAB), 1)
              ).astype(jnp.bfloat16)
        emb_f = jax.lax.dot_general(
            oh, tab, (((1,), (0,)), ((), ())),
            preferred_element_type=jnp.float32)
        acc = jnp.maximum(acc, emb_f)
    x = p.astype(jnp.float32) / 99.0
    cont = jax.lax.dot_general(
        x, w_ref[...], (((1,), (0,)), ((), ())),
        preferred_element_type=jnp.float32) + b_ref[...]
    o_ref[:, :EMB] = acc
    o_ref[:, EMB:] = cont


def kernel(player, embed_table, W_cont, b_cont):
    tab_bf = embed_table.astype(jnp.bfloat16)
    wT = W_cont.T
    b2 = b_cont.reshape(1, EMB)
    out = pl.pallas_call(
        _body,
        grid=(B // BLK,),
        in_specs=[
            pl.BlockSpec((BLK, NFEAT), lambda i: (i, 0)),
            pl.BlockSpec((VOCAB, EMB), lambda i: (0, 0)),
            pl.BlockSpec((NFEAT, EMB), lambda i: (0, 0)),
            pl.BlockSpec((1, EMB), lambda i: (0, 0)),
        ],
        out_specs=pl.BlockSpec((BLK, 2 * EMB), lambda i: (i, 0)),
        out_shape=jax.ShapeDtypeStruct((B, 2 * EMB), jnp.float32),
    )(player, tab_bf, wT, b2)
    return out
